# initial kernel scaffold (unmeasured)
import jax
import jax.numpy as jnp
from jax import lax
from jax.experimental import pallas as pl
from jax.experimental.pallas import tpu as pltpu


def kernel(
    x,
):
    def body(*refs):
        pass

    out_shape = jax.ShapeDtypeStruct(..., jnp.float32)
    return pl.pallas_call(body, out_shape=out_shape)(...)


# baseline (device time: 74682 ns/iter reference)
import jax
import jax.numpy as jnp
from jax import lax
from jax.experimental import pallas as pl
from jax.experimental.pallas import tpu as pltpu

_BLOCK = 128


def kernel(x):
    m, n = x.shape
    out_dtype = jnp.bfloat16
    n_blocks = m // _BLOCK
    assert m % _BLOCK == 0

    def body(x_ref, out_ref, send_row, send_col, halo_row, halo_col,
             send_sems, recv_sems):
        my_x = lax.axis_index("x")
        my_y = lax.axis_index("y")
        x_nbr = (1 - my_x, my_y)
        y_nbr = (my_x, 1 - my_y)

        barrier_sem = pltpu.get_barrier_semaphore()
        for nbr in (x_nbr, y_nbr):
            pl.semaphore_signal(
                barrier_sem, inc=1,
                device_id=nbr, device_id_type=pl.DeviceIdType.MESH,
            )
        pl.semaphore_wait(barrier_sem, 2)

        @pl.when(my_x == 0)
        def _():
            send_row[...] = x_ref[m - 1:m, :]

        @pl.when(my_x == 1)
        def _():
            send_row[...] = x_ref[0:1, :]

        @pl.when(my_y == 0)
        def _():
            send_col[...] = x_ref[:, n - 1:n]

        @pl.when(my_y == 1)
        def _():
            send_col[...] = x_ref[:, 0:1]

        rdma_row = pltpu.make_async_remote_copy(
            src_ref=send_row,
            dst_ref=halo_row,
            send_sem=send_sems.at[0],
            recv_sem=recv_sems.at[0],
            device_id=x_nbr,
            device_id_type=pl.DeviceIdType.MESH,
        )
        rdma_row.start()
        rdma_col = pltpu.make_async_remote_copy(
            src_ref=send_col,
            dst_ref=halo_col,
            send_sem=send_sems.at[1],
            recv_sem=recv_sems.at[1],
            device_id=y_nbr,
            device_id_type=pl.DeviceIdType.MESH,
        )
        rdma_col.start()
        rdma_row.wait()
        rdma_col.wait()

        ri = lax.broadcasted_iota(jnp.int32, (_BLOCK, n), 0)
        ci = lax.broadcasted_iota(jnp.int32, (_BLOCK, n), 1)

        for k in range(n_blocks):
            r0 = k * _BLOCK
            xb = x_ref[r0:r0 + _BLOCK, :]
            prev = halo_row[...] if k == 0 else x_ref[r0 - 1:r0, :]
            nxt = (halo_row[...] if k == n_blocks - 1
                   else x_ref[r0 + _BLOCK:r0 + _BLOCK + 1, :])
            hcb = halo_col[r0:r0 + _BLOCK, :]

            up = jnp.concatenate([prev, xb[:-1, :]], axis=0)
            dn = jnp.concatenate([xb[1:, :], nxt], axis=0)
            s = up + dn
            s = s + jnp.concatenate([hcb, xb[:, :-1]], axis=1)
            s = s + jnp.concatenate([xb[:, 1:], hcb], axis=1)
            acc = 0.5 * xb + 0.125 * s

            mask = ((my_y == 0) & (ci == 0)) | ((my_y == 1) & (ci == n - 1))
            if k == 0:
                mask = mask | ((my_x == 0) & (ri == 0))
            if k == n_blocks - 1:
                mask = mask | ((my_x == 1) & (ri == _BLOCK - 1))
            out_ref[r0:r0 + _BLOCK, :] = jnp.where(mask, xb, acc).astype(out_dtype)

    return pl.pallas_call(
        body,
        out_shape=jax.ShapeDtypeStruct((m, n), out_dtype),
        in_specs=[pl.BlockSpec(memory_space=pltpu.VMEM)],
        out_specs=pl.BlockSpec(memory_space=pltpu.VMEM),
        scratch_shapes=[
            pltpu.VMEM((1, n), x.dtype),
            pltpu.VMEM((m, 1), x.dtype),
            pltpu.VMEM((1, n), x.dtype),
            pltpu.VMEM((m, 1), x.dtype),
            pltpu.SemaphoreType.DMA((2,)),
            pltpu.SemaphoreType.DMA((2,)),
        ],
        compiler_params=pltpu.CompilerParams(
            collective_id=0,
            vmem_limit_bytes=64 * 1024 * 1024,
        ),
    )(x)
